# baseline (speedup 1.0000x reference)
import jax
import jax.numpy as jnp
from jax.experimental import pallas as pl
from jax.experimental.pallas import tpu as pltpu

Q = 16384
D = 768
K = 10000
KPAD = 10240
BQ = 512
BK = 2048
NQ = Q // BQ
NK = KPAD // BK
HB = BK // 2


def _trunc_split(x):
    xh = jax.lax.bitcast_convert_type(
        jax.lax.bitcast_convert_type(x, jnp.uint32) & jnp.uint32(0xFFFF0000),
        jnp.float32)
    return xh.astype(jnp.bfloat16), (x - xh).astype(jnp.bfloat16)


def _nearest_body(f_ref, cth_a, ctl_a, cth_b, ctl_b, ctf_ref, tid_ref,
                  out_ref, bv_ref, bc_ref, csq_ref, code_ref,
                  fnh_ref, fnl_ref, fsq_ref):
    q = pl.program_id(0)
    k = pl.program_id(1)

    @pl.when(k == 0)
    def _():
        f = f_ref[...]
        nrm = jnp.sqrt(jnp.sum(f * f, axis=1, keepdims=True))
        fn = f / jnp.maximum(nrm, 1e-12)
        fh, fl = _trunc_split(fn)
        fnh_ref[...] = fh
        fnl_ref[...] = fl
        fsq_ref[...] = jnp.sum(fn * fn, axis=1, keepdims=True)
        bv_ref[...] = jnp.full((BQ, 1), jnp.inf, jnp.float32)
        bc_ref[...] = jnp.zeros((BQ, 1), jnp.int32)

    @pl.when(q == 0)
    def _():
        c = ctf_ref[...]
        csq_ref[0, pl.ds(k * BK, BK)] = jnp.sum(c * c, axis=0)
        col1 = k * BK + jax.lax.broadcasted_iota(jnp.int32, (1, BK), 1)
        code_ref[0, pl.ds(k * BK, BK)] = (col1 * 16 + tid_ref[0])[0]

    dn = (((1,), (0,)), ((), ()))
    fnh = fnh_ref[...]
    fnl = fnl_ref[...]
    fsq = fsq_ref[...]
    for h, (ch_ref, cl_ref) in enumerate(((cth_a, ctl_a), (cth_b, ctl_b))):
        base = k * BK + h * HB
        hh = jax.lax.dot_general(fnh, ch_ref[...], dn,
                                 preferred_element_type=jnp.float32)
        hl = jax.lax.dot_general(fnh, cl_ref[...], dn,
                                 preferred_element_type=jnp.float32)
        lh = jax.lax.dot_general(fnl, ch_ref[...], dn,
                                 preferred_element_type=jnp.float32)
        dot = (hh + hl) + lh

        d2 = fsq + csq_ref[0, pl.ds(base, HB)][None, :] - 2.0 * dot
        d = jnp.sqrt(jnp.maximum(d2, 0.0))
        col = base + jax.lax.broadcasted_iota(jnp.int32, (BQ, HB), 1)
        d = jnp.where(col < K, d, jnp.inf)

        cmin = jnp.min(d, axis=1, keepdims=True)
        code = code_ref[0, pl.ds(base, HB)][None, :]
        ccode = jnp.min(jnp.where(d == cmin, code, jnp.int32(2**31 - 1)),
                        axis=1, keepdims=True)

        bv = bv_ref[...]
        take = cmin < bv
        bv_ref[...] = jnp.where(take, cmin, bv)
        bc_ref[...] = jnp.where(take, ccode, bc_ref[...])

    @pl.when(k == NK - 1)
    def _():
        out_ref[...] = (bc_ref[...] & 15)[None]


def kernel(features, centroids, task_ids):
    ct = jnp.pad(centroids, ((0, KPAD - K), (0, 0))).T
    cth, ctl = _trunc_split(ct)
    tid = jnp.pad(task_ids, (0, KPAD - K)).reshape(1, 1, KPAD)
    out = pl.pallas_call(
        _nearest_body,
        grid=(NQ, NK),
        in_specs=[
            pl.BlockSpec((BQ, D), lambda q, k: (q, 0)),
            pl.BlockSpec((D, HB), lambda q, k: (0, 2 * k)),
            pl.BlockSpec((D, HB), lambda q, k: (0, 2 * k)),
            pl.BlockSpec((D, HB), lambda q, k: (0, 2 * k + 1)),
            pl.BlockSpec((D, HB), lambda q, k: (0, 2 * k + 1)),
            pl.BlockSpec((D, BK), lambda q, k: (0, jnp.where(q == 0, k, 0))),
            pl.BlockSpec((1, 1, BK), lambda q, k: (0, 0, k)),
        ],
        out_specs=pl.BlockSpec((1, BQ, 1), lambda q, k: (q, 0, 0)),
        out_shape=jax.ShapeDtypeStruct((NQ, BQ, 1), jnp.int32),
        scratch_shapes=[
            pltpu.VMEM((BQ, 1), jnp.float32),
            pltpu.VMEM((BQ, 1), jnp.int32),
            pltpu.VMEM((1, KPAD), jnp.float32),
            pltpu.VMEM((1, KPAD), jnp.int32),
            pltpu.VMEM((BQ, D), jnp.bfloat16),
            pltpu.VMEM((BQ, D), jnp.bfloat16),
            pltpu.VMEM((BQ, 1), jnp.float32),
        ],
    )(features, cth, ctl, cth, ctl, ct, tid)
    return out.reshape(Q)


# speedup vs baseline: 1.0155x; 1.0155x over previous
import jax
import jax.numpy as jnp
from jax.experimental import pallas as pl
from jax.experimental.pallas import tpu as pltpu

Q = 16384
D = 768
K = 10000
KPAD = 10240
BQ = 512
BK = 2048
NQ = Q // BQ
NK = KPAD // BK


def _trunc_split(x):
    xh = jax.lax.bitcast_convert_type(
        jax.lax.bitcast_convert_type(x, jnp.uint32) & jnp.uint32(0xFFFF0000),
        jnp.float32)
    return xh.astype(jnp.bfloat16), (x - xh).astype(jnp.bfloat16)


def _nearest_body(f_ref, cth_ref, ctl_ref, ctf_ref, tid_ref, out_ref,
                  bv_ref, bc_ref, csq_ref, code_ref,
                  fnh_ref, fnl_ref, fsq_ref):
    q = pl.program_id(0)
    k = pl.program_id(1)

    @pl.when(k == 0)
    def _():
        f = f_ref[...]
        nrm = jnp.sqrt(jnp.sum(f * f, axis=1, keepdims=True))
        fn = f / jnp.maximum(nrm, 1e-12)
        fh, fl = _trunc_split(fn)
        fnh_ref[...] = fh
        fnl_ref[...] = fl
        fsq_ref[...] = jnp.sum(fn * fn, axis=1, keepdims=True)
        bv_ref[...] = jnp.full((BQ, 1), jnp.inf, jnp.float32)
        bc_ref[...] = jnp.zeros((BQ, 1), jnp.int32)

    @pl.when(q == 0)
    def _():
        c = ctf_ref[...]
        csq_ref[0, pl.ds(k * BK, BK)] = jnp.sum(c * c, axis=0)
        col1 = k * BK + jax.lax.broadcasted_iota(jnp.int32, (1, BK), 1)
        code_ref[0, pl.ds(k * BK, BK)] = (col1 * 16 + tid_ref[0])[0]

    dn = (((1,), (0,)), ((), ()))
    hh = jax.lax.dot_general(fnh_ref[...], cth_ref[...], dn,
                             preferred_element_type=jnp.float32)
    hl = jax.lax.dot_general(fnh_ref[...], ctl_ref[...], dn,
                             preferred_element_type=jnp.float32)
    lh = jax.lax.dot_general(fnl_ref[...], cth_ref[...], dn,
                             preferred_element_type=jnp.float32)
    dot = (hh + hl) + lh

    d2 = fsq_ref[...] + csq_ref[0, pl.ds(k * BK, BK)][None, :] - 2.0 * dot
    d = jnp.sqrt(jnp.maximum(d2, 0.0))
    col = k * BK + jax.lax.broadcasted_iota(jnp.int32, (BQ, BK), 1)
    d = jnp.where(col < K, d, jnp.inf)

    cmin = jnp.min(d, axis=1, keepdims=True)
    code = code_ref[0, pl.ds(k * BK, BK)][None, :]
    ccode = jnp.min(jnp.where(d == cmin, code, jnp.int32(2**31 - 1)),
                    axis=1, keepdims=True)

    bv = bv_ref[...]
    take = cmin < bv
    bv_ref[...] = jnp.where(take, cmin, bv)
    bc_ref[...] = jnp.where(take, ccode, bc_ref[...])

    @pl.when(k == NK - 1)
    def _():
        out_ref[...] = (bc_ref[...] & 15)[None]


def kernel(features, centroids, task_ids):
    ct = jnp.pad(centroids, ((0, KPAD - K), (0, 0))).T
    cth, ctl = _trunc_split(ct)
    tid = jnp.pad(task_ids, (0, KPAD - K)).reshape(1, 1, KPAD)
    out = pl.pallas_call(
        _nearest_body,
        grid=(NQ, NK),
        in_specs=[
            pl.BlockSpec((BQ, D), lambda q, k: (q, 0)),
            pl.BlockSpec((D, BK), lambda q, k: (0, k)),
            pl.BlockSpec((D, BK), lambda q, k: (0, k)),
            pl.BlockSpec((D, BK), lambda q, k: (0, jnp.where(q == 0, k, 0))),
            pl.BlockSpec((1, 1, BK), lambda q, k: (0, 0, k)),
        ],
        out_specs=pl.BlockSpec((1, BQ, 1), lambda q, k: (q, 0, 0)),
        out_shape=jax.ShapeDtypeStruct((NQ, BQ, 1), jnp.int32),
        scratch_shapes=[
            pltpu.VMEM((BQ, 1), jnp.float32),
            pltpu.VMEM((BQ, 1), jnp.int32),
            pltpu.VMEM((1, KPAD), jnp.float32),
            pltpu.VMEM((1, KPAD), jnp.int32),
            pltpu.VMEM((BQ, D), jnp.bfloat16),
            pltpu.VMEM((BQ, D), jnp.bfloat16),
            pltpu.VMEM((BQ, 1), jnp.float32),
        ],
    )(features, cth, ctl, ct, tid)
    return out.reshape(Q)
